# baseline (speedup 1.0000x reference)
import functools

import jax
import jax.numpy as jnp
from jax import lax
from jax.experimental import pallas as pl
from jax.experimental.pallas import tpu as pltpu
from jax.experimental.pallas import tpu_sc as plsc

B, C, H, W = 64, 192, 32, 32
P = B * H * W

_info = plsc.get_sparse_core_info()
_NC, _NS, _L = _info.num_cores, _info.num_subcores, _info.num_lanes
_NW = _NC * _NS
_PPW = P // _NW
PB = 64
_NBLK = _PPW // PB
_NG = C // _L
_GBUF = 4
_OBUF = 3
_PH = PB // 2


def _body(x_hbm, perm_hbm, out_hbm, perm_v, xbuf, obuf,
          g0, g1, g2, g3, s0, s1, s2):
    wid = lax.axis_index("s") * _NC + lax.axis_index("c")
    base = wid * _PPW
    pltpu.sync_copy(perm_hbm, perm_v)
    perms = tuple(perm_v[pl.ds(g * _L, _L)] for g in range(_NG))
    gsem = (g0, g1, g2, g3)
    ssem = (s0, s1, s2)

    def dma_in(blk):
        i = blk % _GBUF
        return pltpu.async_copy(x_hbm.at[pl.ds(base + blk * PB, PB)],
                                xbuf.at[i], gsem[i])

    def dma_out_half(blk, h):
        i = blk % _OBUF
        return pltpu.async_copy(
            obuf.at[i].at[pl.ds(h * _PH, _PH)],
            out_hbm.at[pl.ds(base + blk * PB + h * _PH, _PH)],
            ssem[i])

    gath = {}
    scat = {}
    for k in range(min(_GBUF, _NBLK)):
        gath[k] = dma_in(k)
    for blk in range(_NBLK):
        i = blk % _GBUF
        o = blk % _OBUF
        gath[blk].wait()
        if blk >= _OBUF:
            for cp in scat[blk - _OBUF]:
                cp.wait()

        def compute_half(h):
            @plsc.parallel_loop(h * _PH, (h + 1) * _PH, unroll=2,
                                carry=perms)
            def step(p, carry):
                row = jnp.full((_L,), p, jnp.int32)
                vals = [plsc.load_gather(xbuf.at[i], [row, carry[g]])
                        for g in range(_NG)]
                for g in range(_NG):
                    obuf[o, p, pl.ds(g * _L, _L)] = vals[g]
                return carry

        compute_half(0)
        first = dma_out_half(blk, 0)
        compute_half(1)
        scat[blk] = (first, dma_out_half(blk, 1))
        if blk + _GBUF < _NBLK:
            gath[blk + _GBUF] = dma_in(blk + _GBUF)
    for k in range(max(0, _NBLK - _OBUF), _NBLK):
        for cp in scat[k]:
            cp.wait()


_shuffle = functools.partial(
    pl.kernel,
    mesh=plsc.VectorSubcoreMesh(core_axis_name="c", subcore_axis_name="s"),
    out_type=jax.ShapeDtypeStruct((P, C), jnp.float32),
    scratch_types=[
        pltpu.VMEM((C,), jnp.int32),
        pltpu.VMEM((_GBUF, PB, C), jnp.float32),
        pltpu.VMEM((_OBUF, PB, C), jnp.float32),
        pltpu.SemaphoreType.DMA,
        pltpu.SemaphoreType.DMA,
        pltpu.SemaphoreType.DMA,
        pltpu.SemaphoreType.DMA,
        pltpu.SemaphoreType.DMA,
        pltpu.SemaphoreType.DMA,
        pltpu.SemaphoreType.DMA,
    ],
    compiler_params=pltpu.CompilerParams(use_tc_tiling_on_sc=True,
                                         needs_layout_passes=False),
)(_body)


def kernel(x, objective, indices):
    x2 = jnp.transpose(x, (0, 2, 3, 1)).reshape(P, C)
    y2 = _shuffle(x2, indices)
    y = jnp.transpose(y2.reshape(B, H, W, C), (0, 3, 1, 2))
    return y, objective


# speedup vs baseline: 1.0553x; 1.0553x over previous
import functools

import jax
import jax.numpy as jnp
from jax import lax
from jax.experimental import pallas as pl
from jax.experimental.pallas import tpu as pltpu
from jax.experimental.pallas import tpu_sc as plsc

B, C, H, W = 64, 192, 32, 32
P = B * H * W

_info = plsc.get_sparse_core_info()
_NC, _NS, _L = _info.num_cores, _info.num_subcores, _info.num_lanes
_NW = _NC * _NS
_PPW = P // _NW
PB = 64
_NBLK = _PPW // PB
_NG = C // _L
_GBUF = 4
_OBUF = 3
_PH = PB // 2


def _body(x_hbm, perm_hbm, out_hbm, perm_v, xbuf, obuf,
          g0, g1, g2, g3, s0, s1, s2):
    wid = lax.axis_index("s") * _NC + lax.axis_index("c")
    base = wid * _PPW
    pltpu.sync_copy(perm_hbm, perm_v)
    perms = tuple(perm_v[pl.ds(g * _L, _L)] for g in range(_NG))
    gsem = (g0, g1, g2, g3)
    ssem = (s0, s1, s2)

    def dma_in(blk):
        i = blk % _GBUF
        return pltpu.async_copy(x_hbm.at[pl.ds(base + blk * PB, PB)],
                                xbuf.at[i], gsem[i])

    def dma_out_full(blk):
        i = blk % _OBUF
        return pltpu.async_copy(
            obuf.at[i],
            out_hbm.at[pl.ds(base + blk * PB, PB)],
            ssem[i])

    gath = {}
    scat = {}
    for k in range(min(_GBUF, _NBLK)):
        gath[k] = dma_in(k)
    for blk in range(_NBLK):
        i = blk % _GBUF
        o = blk % _OBUF
        gath[blk].wait()
        if blk >= _OBUF:
            for cp in scat[blk - _OBUF]:
                cp.wait()

        @plsc.parallel_loop(0, PB, unroll=2, carry=perms)
        def step(p, carry):
            row = jnp.full((_L,), p, jnp.int32)
            vals = [plsc.load_gather(xbuf.at[i], [row, carry[g]])
                    for g in range(_NG)]
            for g in range(_NG):
                obuf[o, p, pl.ds(g * _L, _L)] = vals[g]
            return carry

        scat[blk] = (dma_out_full(blk),)
        if blk + _GBUF < _NBLK:
            gath[blk + _GBUF] = dma_in(blk + _GBUF)
    for k in range(max(0, _NBLK - _OBUF), _NBLK):
        for cp in scat[k]:
            cp.wait()


_shuffle = functools.partial(
    pl.kernel,
    mesh=plsc.VectorSubcoreMesh(core_axis_name="c", subcore_axis_name="s"),
    out_type=jax.ShapeDtypeStruct((P, C), jnp.float32),
    scratch_types=[
        pltpu.VMEM((C,), jnp.int32),
        pltpu.VMEM((_GBUF, PB, C), jnp.float32),
        pltpu.VMEM((_OBUF, PB, C), jnp.float32),
        pltpu.SemaphoreType.DMA,
        pltpu.SemaphoreType.DMA,
        pltpu.SemaphoreType.DMA,
        pltpu.SemaphoreType.DMA,
        pltpu.SemaphoreType.DMA,
        pltpu.SemaphoreType.DMA,
        pltpu.SemaphoreType.DMA,
    ],
    compiler_params=pltpu.CompilerParams(use_tc_tiling_on_sc=True,
                                         needs_layout_passes=False),
)(_body)


def kernel(x, objective, indices):
    x2 = jnp.transpose(x, (0, 2, 3, 1)).reshape(P, C)
    y2 = _shuffle(x2, indices)
    y = jnp.transpose(y2.reshape(B, H, W, C), (0, 3, 1, 2))
    return y, objective


# speedup vs baseline: 1.0767x; 1.0203x over previous
import functools

import jax
import jax.numpy as jnp
from jax import lax
from jax.experimental import pallas as pl
from jax.experimental.pallas import tpu as pltpu
from jax.experimental.pallas import tpu_sc as plsc

B, C, H, W = 64, 192, 32, 32
P = B * H * W

_info = plsc.get_sparse_core_info()
_NC, _NS, _L = _info.num_cores, _info.num_subcores, _info.num_lanes
_NW = _NC * _NS
_PPW = P // _NW
PB = 64
_NBLK = _PPW // PB
_NG = C // _L
_GBUF = 4
_OBUF = 3
_PH = PB // 2


def _body(x_hbm, perm_hbm, out_hbm, perm_v, xbuf, obuf,
          g0, g1, g2, g3, s0, s1, s2):
    wid = lax.axis_index("s") * _NC + lax.axis_index("c")
    base = wid * _PPW
    gsem = (g0, g1, g2, g3)
    ssem = (s0, s1, s2)

    def dma_in(blk):
        i = blk % _GBUF
        return pltpu.async_copy(x_hbm.at[pl.ds(base + blk * PB, PB)],
                                xbuf.at[i], gsem[i])

    def dma_out_full(blk):
        i = blk % _OBUF
        return pltpu.async_copy(
            obuf.at[i],
            out_hbm.at[pl.ds(base + blk * PB, PB)],
            ssem[i])

    gath = {}
    scat = {}
    for k in range(_GBUF - 1):
        gath[k] = dma_in(k)
    pltpu.sync_copy(perm_hbm, perm_v)
    perms = tuple(perm_v[pl.ds(g * _L, _L)] for g in range(_NG))
    for blk in range(_NBLK):
        i = blk % _GBUF
        o = blk % _OBUF
        nxt = blk + _GBUF - 1
        if nxt < _NBLK and nxt not in gath:
            gath[nxt] = dma_in(nxt)
        gath[blk].wait()
        if blk >= _OBUF:
            for cp in scat[blk - _OBUF]:
                cp.wait()

        @plsc.parallel_loop(0, PB, unroll=2, carry=perms)
        def step(p, carry):
            row = jnp.full((_L,), p, jnp.int32)
            vals = [plsc.load_gather(xbuf.at[i], [row, carry[g]])
                    for g in range(_NG)]
            for g in range(_NG):
                obuf[o, p, pl.ds(g * _L, _L)] = vals[g]
            return carry

        scat[blk] = (dma_out_full(blk),)
    for k in range(max(0, _NBLK - _OBUF), _NBLK):
        for cp in scat[k]:
            cp.wait()


_shuffle = functools.partial(
    pl.kernel,
    mesh=plsc.VectorSubcoreMesh(core_axis_name="c", subcore_axis_name="s"),
    out_type=jax.ShapeDtypeStruct((P, C), jnp.float32),
    scratch_types=[
        pltpu.VMEM((C,), jnp.int32),
        pltpu.VMEM((_GBUF, PB, C), jnp.float32),
        pltpu.VMEM((_OBUF, PB, C), jnp.float32),
        pltpu.SemaphoreType.DMA,
        pltpu.SemaphoreType.DMA,
        pltpu.SemaphoreType.DMA,
        pltpu.SemaphoreType.DMA,
        pltpu.SemaphoreType.DMA,
        pltpu.SemaphoreType.DMA,
        pltpu.SemaphoreType.DMA,
    ],
    compiler_params=pltpu.CompilerParams(use_tc_tiling_on_sc=True,
                                         needs_layout_passes=False),
)(_body)


def kernel(x, objective, indices):
    x2 = jnp.transpose(x, (0, 2, 3, 1)).reshape(P, C)
    y2 = _shuffle(x2, indices)
    y = jnp.transpose(y2.reshape(B, H, W, C), (0, 3, 1, 2))
    return y, objective
